# baseline (speedup 1.0000x reference)
import functools

import jax
import jax.numpy as jnp
from jax import lax
from jax.experimental import pallas as pl
from jax.experimental.pallas import tpu as pltpu, tpu_sc as plsc


def _make_sc_gather(V, D, B, H):
    info = plsc.get_sparse_core_info()
    L = info.num_lanes
    NW = info.num_cores * info.num_subcores
    assert B % (8 * NW) == 0 and D % L == 0
    b_per_w = B // NW
    C = 256
    n_chunks = b_per_w // C
    mesh = plsc.VectorSubcoreMesh(core_axis_name="c", subcore_axis_name="s")

    @functools.partial(
        pl.kernel,
        mesh=mesh,
        out_type=jax.ShapeDtypeStruct((B, 2 * D), jnp.float32),
        scratch_types=[
            pltpu.VMEM((b_per_w,), jnp.int32),
            pltpu.VMEM((C,), jnp.int32),
            pltpu.VMEM((C,), jnp.int32),
            pltpu.VMEM((C, 2 * D), jnp.float32),
            pltpu.VMEM((C, 2 * D), jnp.float32),
            pltpu.SemaphoreType.DMA,
        ],
    )
    def emb(labels_hbm, ta_hbm, tb_hbm, out_hbm,
            lab_v, ia_v, ib_v, ba_v, bb_v, sem):
        wid = lax.axis_index("s") * info.num_cores + lax.axis_index("c")
        base = wid * b_per_w
        pltpu.sync_copy(labels_hbm.at[pl.ds(base, b_per_w)], lab_v)

        def chunk_body(c, _):
            def idx_body(g, _):
                vec = lab_v[pl.ds(c * C + g * L, L)]
                k = lax.shift_right_logical(vec, 1)
                ia_v[pl.ds(g * L, L)] = jnp.minimum(k, H - 1)
                ib_v[pl.ds(g * L, L)] = jnp.clip(k - H, 0, H - 1)
                return 0

            lax.fori_loop(0, C // L, idx_body, 0)
            ca = pltpu.async_copy(ta_hbm.at[ia_v], ba_v, sem)
            cb = pltpu.async_copy(tb_hbm.at[ib_v], bb_v, sem)
            ca.wait()
            cb.wait()

            def sel_body(g, _):
                vec = lab_v[pl.ds(c * C + g * L, L)]
                for j in range(L):
                    i = g * L + j
                    lab = vec[j]
                    k = lax.shift_right_logical(lab, 1)
                    odd = lax.rem(lab, 2) != 0
                    hi = k >= H

                    @pl.when(jnp.logical_and(hi, odd))
                    def _():
                        for q in range(D // L):
                            ba_v[i, pl.ds(q * L, L)] = bb_v[
                                i, pl.ds(D + q * L, L)
                            ]

                    @pl.when(jnp.logical_and(hi, jnp.logical_not(odd)))
                    def _():
                        for q in range(D // L):
                            ba_v[i, pl.ds(q * L, L)] = bb_v[i, pl.ds(q * L, L)]

                    @pl.when(jnp.logical_and(jnp.logical_not(hi), odd))
                    def _():
                        for q in range(D // L):
                            ba_v[i, pl.ds(q * L, L)] = ba_v[
                                i, pl.ds(D + q * L, L)
                            ]

                return 0

            lax.fori_loop(0, C // L, sel_body, 0)
            pltpu.sync_copy(ba_v, out_hbm.at[pl.ds(base + c * C, C)])
            return 0

        lax.fori_loop(0, n_chunks, chunk_body, 0)

    return emb


def kernel(labels, embedding_table):
    B = labels.shape[0]
    V, D = embedding_table.shape
    H = V // 4
    emb = _make_sc_gather(V, D, B, H)
    table2 = embedding_table.reshape(V // 2, 2 * D)
    ta = table2[:H]
    tb = table2[H:]
    out2 = emb(labels.astype(jnp.int32), ta, tb)
    return out2[:, :D]
